# baseline (speedup 1.0000x reference)
import jax
import jax.numpy as jnp
from jax.experimental import pallas as pl
from jax.experimental.pallas import tpu as pltpu

_SBLK = 2048


def _pe_add_body(len_ref, x_ref, pe_ref, se_ref, o_ref, t0_ref):
    b = pl.program_id(1)
    D = pe_ref.shape[1]
    rows = jax.lax.broadcasted_iota(jnp.int32, (_SBLK, 1), 0)

    @pl.when(b == 0)
    def _():
        pe_shift = jnp.concatenate(
            [jnp.zeros((1, D), pe_ref.dtype), pe_ref[...]], axis=0)
        t0_ref[...] = jnp.where(rows != 0,
                                pe_shift + se_ref[0, :][None, :], 0.0)

    len_b = len_ref[b]
    dse = (se_ref[1, :] - se_ref[0, :])[None, :]
    delta = jnp.where(rows <= len_b, jnp.zeros_like(dse), dse)
    o_ref[0] = x_ref[0] + (t0_ref[...] + delta)


def kernel(x, length, pe, se):
    B, S, D = x.shape
    grid_spec = pltpu.PrefetchScalarGridSpec(
        num_scalar_prefetch=1,
        grid=(S // _SBLK, B),
        in_specs=[
            pl.BlockSpec((1, _SBLK, D), lambda si, b, len_ref: (b, si, 0)),
            pl.BlockSpec((S - 1, D), lambda si, b, len_ref: (0, 0)),
            pl.BlockSpec((2, D), lambda si, b, len_ref: (0, 0)),
        ],
        out_specs=pl.BlockSpec((1, _SBLK, D), lambda si, b, len_ref: (b, si, 0)),
        scratch_shapes=[pltpu.VMEM((_SBLK, D), x.dtype)],
    )
    return pl.pallas_call(
        _pe_add_body,
        grid_spec=grid_spec,
        out_shape=jax.ShapeDtypeStruct((B, S, D), x.dtype),
    )(length.astype(jnp.int32), x, pe.reshape(S - 1, D), se)


# speedup vs baseline: 1.1877x; 1.1877x over previous
import jax
import jax.numpy as jnp
from jax.experimental import pallas as pl
from jax.experimental.pallas import tpu as pltpu

_SBLK = 2048


def _pe_add_body(len_ref, x_ref, pe_ref, se_ref, o_ref):
    b = pl.program_id(1)
    rows = jax.lax.broadcasted_iota(jnp.int32, (_SBLK, 1), 0)
    len_b = len_ref[b]
    se_sel = jnp.where(rows <= len_b, se_ref[0, :][None, :], se_ref[1, :][None, :])
    se_sel = jnp.where(rows == 0, jnp.zeros_like(se_sel), se_sel)
    pe_shift = jnp.concatenate(
        [jnp.zeros((1, pe_ref.shape[2]), pe_ref.dtype), pe_ref[0]], axis=0)
    o_ref[0] = x_ref[0] + pe_shift + se_sel


def kernel(x, length, pe, se):
    B, S, D = x.shape
    grid_spec = pltpu.PrefetchScalarGridSpec(
        num_scalar_prefetch=1,
        grid=(S // _SBLK, B),
        in_specs=[
            pl.BlockSpec((1, _SBLK, D), lambda si, b, len_ref: (b, si, 0)),
            pl.BlockSpec((1, S - 1, D), lambda si, b, len_ref: (0, 0, 0)),
            pl.BlockSpec((2, D), lambda si, b, len_ref: (0, 0)),
        ],
        out_specs=pl.BlockSpec((1, _SBLK, D), lambda si, b, len_ref: (b, si, 0)),
    )
    return pl.pallas_call(
        _pe_add_body,
        grid_spec=grid_spec,
        out_shape=jax.ShapeDtypeStruct((B, S, D), x.dtype),
    )(length.astype(jnp.int32), x, pe, se)
